# baseline (speedup 1.0000x reference)
import functools

import jax
import jax.numpy as jnp
from jax import lax
from jax.experimental import pallas as pl
from jax.experimental.pallas import tpu as pltpu
from jax.experimental.pallas import tpu_sc as plsc

VOCAB = 1000000
EMB = 64
MAXLEN = 200
BATCH = 4096

NUM_CORES = 2
NUM_SUBCORES = 16
NW = NUM_CORES * NUM_SUBCORES
BTILE = BATCH // NW
LTILES = MAXLEN // 8
DTILES = EMB // 8
OTILE = EMB * BTILE
NB = 4
LA = 3


def _make_kernel():
    mesh = plsc.VectorSubcoreMesh(core_axis_name="c", subcore_axis_name="s")

    @functools.partial(
        pl.kernel,
        mesh=mesh,
        out_type=jax.ShapeDtypeStruct((MAXLEN, DTILES, NW, 8 * BTILE),
                                      jnp.float32),
        scratch_types=[
            pltpu.VMEM((LTILES, 8 * BTILE), jnp.int32),
            pltpu.VMEM((NB, BTILE, EMB), jnp.float32),
            pltpu.VMEM((2, DTILES, 8 * BTILE), jnp.float32),
            pltpu.VMEM((MAXLEN, EMB), jnp.float32),
            pltpu.SemaphoreType.DMA,
            pltpu.SemaphoreType.DMA((NB,)),
            pltpu.SemaphoreType.DMA((2,)),
        ],
        compiler_params=pltpu.CompilerParams(use_tc_tiling_on_sc=False,
                                             needs_layout_passes=False),
    )
    def emb_kernel(x_hbm, tok_hbm, pos_hbm, out_hbm, idx_v, tok_v, obuf_v,
                   pos_v, psem, gsem, osem):
        wid = lax.axis_index("s") * NUM_CORES + lax.axis_index("c")

        pltpu.async_copy(pos_hbm, pos_v, psem)
        pltpu.async_copy(x_hbm.at[pl.ds(0, LTILES), wid], idx_v, psem)
        pltpu.make_async_copy(pos_hbm, pos_v, psem).wait()
        pltpu.make_async_copy(x_hbm.at[pl.ds(0, LTILES), wid], idx_v,
                              psem).wait()

        def gather_desc(l):
            lt = l // 8
            li = lax.rem(l, 8)
            s = lax.rem(l, NB)
            return (tok_hbm.at[idx_v.at[lt, pl.ds(li * BTILE, BTILE)]],
                    tok_v.at[s], gsem.at[s])

        def out_descs(l, p):
            return [(obuf_v.at[p], out_hbm.at[l, pl.ds(0, DTILES), wid],
                     osem.at[p])]

        for l0 in range(LA):
            pltpu.async_copy(*gather_desc(l0))

        lane = lax.iota(jnp.int32, 16)
        dtv = [lane // 8 + 2 * j for j in range(EMB // 16)]
        colbase = (lane % 8) * BTILE

        def pos_body(l, carry):
            @pl.when(l + LA < MAXLEN)
            def _issue_ahead():
                pltpu.async_copy(*gather_desc(l + LA))

            s = lax.rem(l, NB)
            p = lax.rem(l, 2)
            pltpu.make_async_copy(*gather_desc(l)).wait()

            @pl.when(l >= 2)
            def _drain_out():
                for d in out_descs(l - 2, p):
                    pltpu.make_async_copy(*d).wait()

            tv = tok_v.at[s]
            pvec = [pos_v[l, pl.ds(j * 16, 16)] for j in range(EMB // 16)]
            psplat = jnp.full((16,), p, jnp.int32)

            @plsc.parallel_loop(0, BTILE, 1, unroll=8)
            def b_body(b):
                col = colbase + b
                for j in range(EMB // 16):
                    v = tv[b, pl.ds(j * 16, 16)] + pvec[j]
                    plsc.store_scatter(obuf_v, [psplat, dtv[j], col], v)

            for d in out_descs(l, p):
                pltpu.async_copy(*d)
            return carry

        lax.fori_loop(0, MAXLEN, pos_body, 0)
        for ll in range(MAXLEN - 2, MAXLEN):
            for d in out_descs(ll, lax.rem(ll, 2)):
                pltpu.make_async_copy(*d).wait()

    return emb_kernel


_EMB_KERNEL = _make_kernel()


def kernel(x, token_table, pos_table):
    xv = (x.astype(jnp.int32).T
          .reshape(LTILES, 8, NW, BTILE)
          .transpose(0, 2, 1, 3)
          .reshape(LTILES, NW, 8 * BTILE))
    o4 = _EMB_KERNEL(xv, token_table, pos_table)
    return (o4.reshape(MAXLEN, DTILES, NW, 8, BTILE)
            .transpose(2, 4, 0, 1, 3)
            .reshape(BATCH, MAXLEN, EMB))


# speedup vs baseline: 1.0041x; 1.0041x over previous
import functools

import jax
import jax.numpy as jnp
from jax import lax
from jax.experimental import pallas as pl
from jax.experimental.pallas import tpu as pltpu
from jax.experimental.pallas import tpu_sc as plsc

VOCAB = 1000000
EMB = 64
MAXLEN = 200
BATCH = 4096

NUM_CORES = 2
NUM_SUBCORES = 16
NW = NUM_CORES * NUM_SUBCORES
BTILE = BATCH // NW
LTILES = MAXLEN // 8
DTILES = EMB // 8
NGRP = EMB // 16
GRPB = 2 * 8 * BTILE
NB = 6
LA = 5


def _make_kernel():
    mesh = plsc.VectorSubcoreMesh(core_axis_name="c", subcore_axis_name="s")

    @functools.partial(
        pl.kernel,
        mesh=mesh,
        out_type=jax.ShapeDtypeStruct((MAXLEN, DTILES, NW, 8 * BTILE),
                                      jnp.float32),
        scratch_types=[
            pltpu.VMEM((LTILES, 8 * BTILE), jnp.int32),
            pltpu.VMEM((NB, BTILE, EMB), jnp.float32),
            pltpu.VMEM((2 * GRPB,), jnp.float32),
            pltpu.VMEM((2 * GRPB,), jnp.float32),
            pltpu.VMEM((2 * GRPB,), jnp.float32),
            pltpu.VMEM((2 * GRPB,), jnp.float32),
            pltpu.VMEM((MAXLEN, EMB), jnp.float32),
            pltpu.SemaphoreType.DMA,
            pltpu.SemaphoreType.DMA((NB,)),
            pltpu.SemaphoreType.DMA((2,)),
        ],
        compiler_params=pltpu.CompilerParams(use_tc_tiling_on_sc=False,
                                             needs_layout_passes=False),
    )
    def emb_kernel(x_hbm, tok_hbm, pos_hbm, out_hbm, idx_v, tok_v,
                   ob0, ob1, ob2, ob3, pos_v, psem, gsem, osem):
        obufs = (ob0, ob1, ob2, ob3)
        wid = lax.axis_index("s") * NUM_CORES + lax.axis_index("c")

        pltpu.async_copy(pos_hbm, pos_v, psem)
        pltpu.async_copy(x_hbm.at[pl.ds(0, LTILES), wid], idx_v, psem)
        pltpu.make_async_copy(pos_hbm, pos_v, psem).wait()
        pltpu.make_async_copy(x_hbm.at[pl.ds(0, LTILES), wid], idx_v,
                              psem).wait()

        def gather_desc(l):
            lt = l // 8
            li = lax.rem(l, 8)
            s = lax.rem(l, NB)
            return (tok_hbm.at[idx_v.at[lt, pl.ds(li * BTILE, BTILE)]],
                    tok_v.at[s], gsem.at[s])

        def out_descs(l, p):
            descs = []
            for j in range(NGRP):
                for k in range(2):
                    src = obufs[j].at[pl.ds(p * GRPB + k * 8 * BTILE,
                                            8 * BTILE)]
                    descs.append((src, out_hbm.at[l, 2 * j + k, wid],
                                  osem.at[p]))
            return descs

        for l0 in range(LA):
            pltpu.async_copy(*gather_desc(l0))

        lane = lax.iota(jnp.int32, 16)
        fbase = (lane // 8) * (8 * BTILE) + (lane % 8) * BTILE

        def pos_body(l, carry):
            @pl.when(l + LA < MAXLEN)
            def _issue_ahead():
                pltpu.async_copy(*gather_desc(l + LA))

            s = lax.rem(l, NB)
            p = lax.rem(l, 2)
            pltpu.make_async_copy(*gather_desc(l)).wait()

            @pl.when(l >= 2)
            def _drain_out():
                for d in out_descs(l - 2, p):
                    pltpu.make_async_copy(*d).wait()

            tv = tok_v.at[s]
            pvec = [pos_v[l, pl.ds(j * 16, 16)] for j in range(NGRP)]
            fidx0 = fbase + p * GRPB

            @plsc.parallel_loop(0, BTILE, 1, unroll=8, carry=fidx0)
            def b_body(b, fidx):
                for j in range(NGRP):
                    v = tv[b, pl.ds(j * 16, 16)] + pvec[j]
                    plsc.store_scatter(obufs[j], [fidx], v)
                return fidx + 1

            for d in out_descs(l, p):
                pltpu.async_copy(*d)
            return carry

        lax.fori_loop(0, MAXLEN, pos_body, 0)
        for ll in range(MAXLEN - 2, MAXLEN):
            for d in out_descs(ll, lax.rem(ll, 2)):
                pltpu.make_async_copy(*d).wait()

    return emb_kernel


_EMB_KERNEL = _make_kernel()


def kernel(x, token_table, pos_table):
    xv = (x.astype(jnp.int32).T
          .reshape(LTILES, 8, NW, BTILE)
          .transpose(0, 2, 1, 3)
          .reshape(LTILES, NW, 8 * BTILE))
    o4 = _EMB_KERNEL(xv, token_table, pos_table)
    return (o4.reshape(MAXLEN, DTILES, NW, 8, BTILE)
            .transpose(2, 4, 0, 1, 3)
            .reshape(BATCH, MAXLEN, EMB))


# speedup vs baseline: 1.7586x; 1.7514x over previous
import functools

import jax
import jax.numpy as jnp
from jax import lax
from jax.experimental import pallas as pl
from jax.experimental.pallas import tpu as pltpu
from jax.experimental.pallas import tpu_sc as plsc

VOCAB = 1000000
EMB = 64
MAXLEN = 200
BATCH = 4096

NUM_CORES = 2
NUM_SUBCORES = 16
NW = NUM_CORES * NUM_SUBCORES
BTILE = BATCH // NW
LTILES = MAXLEN // 8
DTILES = EMB // 8
NGRP = EMB // 16
PITCH = BTILE + 1
NB = 6
LA = 5


def _make_kernel():
    mesh = plsc.VectorSubcoreMesh(core_axis_name="c", subcore_axis_name="s")

    @functools.partial(
        pl.kernel,
        mesh=mesh,
        out_type=jax.ShapeDtypeStruct((MAXLEN, DTILES, NW, 8, BTILE),
                                      jnp.float32),
        scratch_types=[
            pltpu.VMEM((LTILES, 8 * BTILE), jnp.int32),
            pltpu.VMEM((NB, BTILE, EMB), jnp.float32),
            pltpu.VMEM((2 * 2 * 8, PITCH), jnp.float32),
            pltpu.VMEM((2 * 2 * 8, PITCH), jnp.float32),
            pltpu.VMEM((2 * 2 * 8, PITCH), jnp.float32),
            pltpu.VMEM((2 * 2 * 8, PITCH), jnp.float32),
            pltpu.VMEM((MAXLEN, EMB), jnp.float32),
            pltpu.SemaphoreType.DMA,
            pltpu.SemaphoreType.DMA((NB,)),
            pltpu.SemaphoreType.DMA((2,)),
        ],
        compiler_params=pltpu.CompilerParams(use_tc_tiling_on_sc=False,
                                             needs_layout_passes=False),
    )
    def emb_kernel(x_hbm, tok_hbm, pos_hbm, out_hbm, idx_v, tok_v,
                   ob0, ob1, ob2, ob3, pos_v, psem, gsem, osem):
        obufs = (ob0, ob1, ob2, ob3)
        wid = lax.axis_index("s") * NUM_CORES + lax.axis_index("c")

        pltpu.async_copy(pos_hbm, pos_v, psem)
        pltpu.async_copy(x_hbm.at[pl.ds(0, LTILES), wid], idx_v, psem)
        pltpu.make_async_copy(pos_hbm, pos_v, psem).wait()
        pltpu.make_async_copy(x_hbm.at[pl.ds(0, LTILES), wid], idx_v,
                              psem).wait()

        def gather_desc(l):
            lt = l // 8
            li = lax.rem(l, 8)
            s = lax.rem(l, NB)
            return (tok_hbm.at[idx_v.at[lt, pl.ds(li * BTILE, BTILE)]],
                    tok_v.at[s], gsem.at[s])

        def out_descs(l, p):
            descs = []
            for j in range(NGRP):
                for k in range(2):
                    src = obufs[j].at[pl.ds(p * 16 + k * 8, 8),
                                      pl.ds(0, BTILE)]
                    descs.append((src, out_hbm.at[l, 2 * j + k, wid],
                                  osem.at[p]))
            return descs

        for l0 in range(LA):
            pltpu.async_copy(*gather_desc(l0))

        lane = lax.iota(jnp.int32, 16)
        zero16 = lane * 0

        def pos_body(l, carry):
            @pl.when(l + LA < MAXLEN)
            def _issue_ahead():
                pltpu.async_copy(*gather_desc(l + LA))

            s = lax.rem(l, NB)
            p = lax.rem(l, 2)
            pltpu.make_async_copy(*gather_desc(l)).wait()

            @pl.when(l >= 2)
            def _drain_out():
                for d in out_descs(l - 2, p):
                    pltpu.make_async_copy(*d).wait()

            tv = tok_v.at[s]
            pvec = [pos_v[l, pl.ds(j * 16, 16)] for j in range(NGRP)]
            rowv = lane + p * 16

            @plsc.parallel_loop(0, BTILE, 1, unroll=8, carry=zero16)
            def b_body(b, colv):
                for j in range(NGRP):
                    v = tv[b, pl.ds(j * 16, 16)] + pvec[j]
                    plsc.store_scatter(obufs[j], [rowv, colv], v)
                return colv + 1

            for d in out_descs(l, p):
                pltpu.async_copy(*d)
            return carry

        lax.fori_loop(0, MAXLEN, pos_body, 0)
        for ll in range(MAXLEN - 2, MAXLEN):
            for d in out_descs(ll, lax.rem(ll, 2)):
                pltpu.make_async_copy(*d).wait()

    return emb_kernel


_EMB_KERNEL = _make_kernel()


def kernel(x, token_table, pos_table):
    xv = (x.astype(jnp.int32).T
          .reshape(LTILES, 8, NW, BTILE)
          .transpose(0, 2, 1, 3)
          .reshape(LTILES, NW, 8 * BTILE))
    o5 = _EMB_KERNEL(xv, token_table, pos_table)
    return (o5.transpose(2, 4, 0, 1, 3)
            .reshape(BATCH, MAXLEN, EMB))
